# baseline (speedup 1.0000x reference)
import jax
import jax.numpy as jnp
from jax.experimental import pallas as pl
from jax.experimental.pallas import tpu as pltpu

_BLK = 8192
_S = 2
_NEG = -1e30


def _gate_chunk(x, w1, b1, w2, b2col):
    h = jnp.dot(x, w1, preferred_element_type=jnp.float32) + b1
    h = 0.5 * h * (1.0 + jax.lax.erf(h * 0.7071067811865476))
    lt = jax.lax.dot_general(
        w2, h, (((0,), (1,)), ((), ())),
        preferred_element_type=jnp.float32,
    ) + b2col
    m = jnp.max(lt, axis=0, keepdims=True)
    e = jnp.exp(lt - m)
    s = jnp.sum(e, axis=0, keepdims=True)
    g = e / s
    g0 = g[0:1, :]
    g1 = g[1:2, :]
    g2 = g[2:3, :]
    excl2 = (g2 <= g0) & (g2 <= g1)
    excl1 = (~excl2) & (g1 <= g0) & (g1 < g2)
    excl0 = (~excl2) & (~excl1)
    ones = jnp.ones_like(g0)
    zeros = jnp.zeros_like(g0)
    mt = jnp.concatenate(
        [
            jnp.where(excl0, zeros, ones),
            jnp.where(excl1, zeros, ones),
            jnp.where(excl2, zeros, ones),
        ],
        axis=0,
    )
    gt = g[0:3, :] * mt
    gt = gt / (jnp.sum(gt, axis=0, keepdims=True) + 1e-8)
    return gt.T, mt.T


def _gate_body(*refs):
    xs = refs[:_S]
    w1 = refs[_S][...]
    b1 = refs[_S + 1][...]
    w2 = refs[_S + 2][...]
    b2col = refs[_S + 3][...].T
    gated_ref, mask_ref = refs[_S + 4], refs[_S + 5]
    for j in range(_S):
        gated, mask = _gate_chunk(xs[j][...], w1, b1, w2, b2col)
        gated_ref[pl.ds(j * _BLK, _BLK), :] = gated
        mask_ref[pl.ds(j * _BLK, _BLK), :] = mask


def kernel(combined_pooled_feat, W1, b1, W2, b2):
    n, d_in = combined_pooled_feat.shape
    d_h = W1.shape[1]
    n_out = W2.shape[1]
    super_blk = _S * _BLK
    grid = (n // super_blk,)

    def mk_x_spec(j):
        return pl.BlockSpec((_BLK, d_in), lambda i, j=j: (i * _S + j, 0))

    gated, mask = pl.pallas_call(
        _gate_body,
        grid=grid,
        in_specs=[mk_x_spec(j) for j in range(_S)] + [
            pl.BlockSpec((d_in, d_h), lambda i: (0, 0)),
            pl.BlockSpec((1, d_h), lambda i: (0, 0)),
            pl.BlockSpec((d_h, n_out), lambda i: (0, 0)),
            pl.BlockSpec((1, n_out), lambda i: (0, 0)),
        ],
        out_specs=[
            pl.BlockSpec((super_blk, n_out), lambda i: (i, 0)),
            pl.BlockSpec((super_blk, n_out), lambda i: (i, 0)),
        ],
        out_shape=[
            jax.ShapeDtypeStruct((n, n_out), jnp.float32),
            jax.ShapeDtypeStruct((n, n_out), jnp.float32),
        ],
        compiler_params=pltpu.CompilerParams(
            dimension_semantics=("arbitrary",),
        ),
    )(*([combined_pooled_feat] * _S), W1, b1.reshape(1, d_h), W2, b2.reshape(1, n_out))
    return (gated, mask)


# speedup vs baseline: 1.0931x; 1.0931x over previous
import jax
import jax.numpy as jnp
from jax.experimental import pallas as pl
from jax.experimental.pallas import tpu as pltpu

_BLK = 4096
_S = 2
_NEG = -1e30


def _gate_chunk(x, w1, b1, w2, b2col):
    h = jnp.dot(x, w1, preferred_element_type=jnp.float32) + b1
    h = 0.5 * h * (1.0 + jax.lax.erf(h * 0.7071067811865476))
    lt = jax.lax.dot_general(
        w2, h, (((0,), (1,)), ((), ())),
        preferred_element_type=jnp.float32,
    ) + b2col
    m = jnp.max(lt, axis=0, keepdims=True)
    e = jnp.exp(lt - m)
    s = jnp.sum(e, axis=0, keepdims=True)
    g = e / s
    g0 = g[0:1, :]
    g1 = g[1:2, :]
    g2 = g[2:3, :]
    excl2 = (g2 <= g0) & (g2 <= g1)
    excl1 = (~excl2) & (g1 <= g0) & (g1 < g2)
    excl0 = (~excl2) & (~excl1)
    ones = jnp.ones_like(g0)
    zeros = jnp.zeros_like(g0)
    mt = jnp.concatenate(
        [
            jnp.where(excl0, zeros, ones),
            jnp.where(excl1, zeros, ones),
            jnp.where(excl2, zeros, ones),
        ],
        axis=0,
    )
    gt = g[0:3, :] * mt
    gt = gt / (jnp.sum(gt, axis=0, keepdims=True) + 1e-8)
    return gt.T, mt.T


def _gate_body(*refs):
    xs = refs[:_S]
    w1 = refs[_S][...]
    b1 = refs[_S + 1][...]
    w2 = refs[_S + 2][...]
    b2col = refs[_S + 3][...].T
    gated_ref, mask_ref = refs[_S + 4], refs[_S + 5]
    for j in range(_S):
        gated, mask = _gate_chunk(xs[j][...], w1, b1, w2, b2col)
        gated_ref[pl.ds(j * _BLK, _BLK), :] = gated
        mask_ref[pl.ds(j * _BLK, _BLK), :] = mask


def kernel(combined_pooled_feat, W1, b1, W2, b2):
    n, d_in = combined_pooled_feat.shape
    d_h = W1.shape[1]
    n_out = W2.shape[1]
    super_blk = _S * _BLK
    grid = (n // super_blk,)

    def mk_x_spec(j):
        return pl.BlockSpec((_BLK, d_in), lambda i, j=j: (i * _S + j, 0))

    gated, mask = pl.pallas_call(
        _gate_body,
        grid=grid,
        in_specs=[mk_x_spec(j) for j in range(_S)] + [
            pl.BlockSpec((d_in, d_h), lambda i: (0, 0)),
            pl.BlockSpec((1, d_h), lambda i: (0, 0)),
            pl.BlockSpec((d_h, n_out), lambda i: (0, 0)),
            pl.BlockSpec((1, n_out), lambda i: (0, 0)),
        ],
        out_specs=[
            pl.BlockSpec((super_blk, n_out), lambda i: (i, 0)),
            pl.BlockSpec((super_blk, n_out), lambda i: (i, 0)),
        ],
        out_shape=[
            jax.ShapeDtypeStruct((n, n_out), jnp.float32),
            jax.ShapeDtypeStruct((n, n_out), jnp.float32),
        ],
        compiler_params=pltpu.CompilerParams(
            dimension_semantics=("arbitrary",),
        ),
    )(*([combined_pooled_feat] * _S), W1, b1.reshape(1, d_h), W2, b2.reshape(1, n_out))
    return (gated, mask)
